# baseline (speedup 1.0000x reference)
import functools

import jax
import jax.numpy as jnp
from jax import lax
from jax.experimental import pallas as pl
from jax.experimental.pallas import tpu as pltpu
from jax.experimental.pallas import tpu_sc as plsc


@functools.partial(jax.jit, static_argnames=("ch",))
def _lookup(idx, table, ch):
    B = idx.shape[0]
    _, D = table.shape
    info = plsc.get_sparse_core_info()
    nc, ns = info.num_cores, info.num_subcores
    nw = nc * ns
    b_per_w = B // nw
    n_ch = b_per_w // ch

    mesh = plsc.VectorSubcoreMesh(core_axis_name="c", subcore_axis_name="s")

    @functools.partial(
        pl.kernel,
        mesh=mesh,
        out_type=jax.ShapeDtypeStruct((B, D), jnp.float32),
        compiler_params=pltpu.CompilerParams(use_tc_tiling_on_sc=False),
        scratch_types=[
            pltpu.VMEM((ch,), jnp.int32),
            pltpu.VMEM((ch, D), jnp.float32),
            pltpu.SemaphoreType.DMA,
        ],
    )
    def k(idx_hbm, table_hbm, out_hbm, idx_v, rows_v, sem):
        wid = lax.axis_index("s") * nc + lax.axis_index("c")
        base = wid * b_per_w

        def body(i, carry):
            off = base + i * ch
            pltpu.sync_copy(idx_hbm.at[pl.ds(off, ch)], idx_v)
            pltpu.async_copy(table_hbm.at[idx_v], rows_v, sem).wait()
            pltpu.sync_copy(rows_v, out_hbm.at[pl.ds(off, ch)])
            return carry

        lax.fori_loop(0, n_ch, body, 0)

    return k(idx, table)


def kernel(input, table):
    b0, b1 = input.shape
    d = table.shape[1]
    idx = input.reshape(-1).astype(jnp.int32)
    out = _lookup(idx, table, 128)
    return out.reshape(b0, b1, d)


# speedup vs baseline: 1.1242x; 1.1242x over previous
import functools

import jax
import jax.numpy as jnp
from jax import lax
from jax.experimental import pallas as pl
from jax.experimental.pallas import tpu as pltpu
from jax.experimental.pallas import tpu_sc as plsc


@functools.partial(jax.jit, static_argnames=("ch", "nbuf"))
def _lookup(idx3, table, ch, nbuf):
    nw, n_ch, _ = idx3.shape
    B = nw * n_ch * ch
    _, D = table.shape
    info = plsc.get_sparse_core_info()
    nc = info.num_cores
    b_per_w = n_ch * ch
    n_groups = n_ch // nbuf
    assert n_groups * nbuf == n_ch

    mesh = plsc.VectorSubcoreMesh(core_axis_name="c", subcore_axis_name="s")

    @functools.partial(
        pl.kernel,
        mesh=mesh,
        out_type=jax.ShapeDtypeStruct((B, D), jnp.float32),
        compiler_params=pltpu.CompilerParams(use_tc_tiling_on_sc=False),
        scratch_types=(
            [pltpu.VMEM((n_ch, ch), jnp.int32),
             pltpu.VMEM((nbuf, ch, D), jnp.float32)]
            + [pltpu.SemaphoreType.DMA] * (2 * nbuf)
        ),
    )
    def k(idx_hbm, table_hbm, out_hbm, idx_v, rows_v, *sems):
        g_sems, s_sems = sems[:nbuf], sems[nbuf:]
        wid = lax.axis_index("s") * nc + lax.axis_index("c")
        base = wid * b_per_w

        def gather(b, j):
            return pltpu.make_async_copy(
                table_hbm.at[idx_v.at[j]], rows_v.at[b], g_sems[b])

        def store(b, c):
            return pltpu.make_async_copy(
                rows_v.at[b], out_hbm.at[pl.ds(base + c * ch, ch)], s_sems[b])

        pltpu.sync_copy(idx_hbm.at[wid], idx_v)
        for b in range(nbuf):
            gather(b, b).start()

        def body(g, carry):
            c0 = g * nbuf
            for b in range(nbuf):
                gather(b, c0 + b).wait()
                store(b, c0 + b).start()
            for b in range(nbuf):
                cn = c0 + b + nbuf

                @pl.when(cn < n_ch)
                def _():
                    store(b, 0).wait()
                    gather(b, cn).start()

            return carry

        lax.fori_loop(0, n_groups, body, 0)
        for b in range(nbuf):
            store(b, 0).wait()

    return k(idx3, table)


def kernel(input, table):
    b0, b1 = input.shape
    d = table.shape[1]
    ch, nbuf = 512, 2
    idx = input.reshape(-1).astype(jnp.int32)
    nw = 32
    n_ch = idx.shape[0] // (nw * ch)
    out = _lookup(idx.reshape(nw, n_ch, ch), table, ch, nbuf)
    return out.reshape(b0, b1, d)


# speedup vs baseline: 1.2435x; 1.1061x over previous
import functools

import jax
import jax.numpy as jnp
from jax import lax
from jax.experimental import pallas as pl
from jax.experimental.pallas import tpu as pltpu
from jax.experimental.pallas import tpu_sc as plsc
from jax.experimental import layout as jlayout


@functools.partial(jax.jit, static_argnames=("ch", "nbuf"))
def _lookup(idx3, table, ch, nbuf):
    nw, n_ch, _ = idx3.shape
    B = nw * n_ch * ch
    _, D = table.shape
    info = plsc.get_sparse_core_info()
    nc = info.num_cores
    b_per_w = n_ch * ch
    n_groups = n_ch // nbuf
    assert n_groups * nbuf == n_ch

    mesh = plsc.VectorSubcoreMesh(core_axis_name="c", subcore_axis_name="s")

    @functools.partial(
        pl.kernel,
        mesh=mesh,
        out_type=jax.ShapeDtypeStruct((B, D), jnp.float32),
        compiler_params=pltpu.CompilerParams(use_tc_tiling_on_sc=False),
        scratch_types=(
            [pltpu.VMEM((n_ch, ch), jnp.int32),
             pltpu.VMEM((nbuf, ch, D), jnp.float32)]
            + [pltpu.SemaphoreType.DMA] * (2 * nbuf)
        ),
    )
    def k(idx_hbm, table_hbm, out_hbm, idx_v, rows_v, *sems):
        g_sems, s_sems = sems[:nbuf], sems[nbuf:]
        wid = lax.axis_index("s") * nc + lax.axis_index("c")
        base = wid * b_per_w

        def gather(b, j):
            return pltpu.make_async_copy(
                table_hbm.at[idx_v.at[j]], rows_v.at[b], g_sems[b])

        def store(b, c):
            return pltpu.make_async_copy(
                rows_v.at[b], out_hbm.at[pl.ds(base + c * ch, ch)], s_sems[b])

        pltpu.sync_copy(idx_hbm.at[wid], idx_v)
        for b in range(nbuf):
            gather(b, b).start()

        def body(g, carry):
            c0 = g * nbuf
            for b in range(nbuf):
                gather(b, c0 + b).wait()
                store(b, c0 + b).start()
            for b in range(nbuf):
                cn = c0 + b + nbuf

                @pl.when(cn < n_ch)
                def _():
                    store(b, 0).wait()
                    gather(b, cn).start()

            return carry

        lax.fori_loop(0, n_groups, body, 0)
        for b in range(nbuf):
            store(b, 0).wait()

    return k(idx3, table)


def kernel(input, table):
    b0, b1 = input.shape
    d = table.shape[1]
    ch, nbuf = 512, 2
    idx = input.reshape(-1).astype(jnp.int32)
    nw = 32
    n_ch = idx.shape[0] // (nw * ch)
    out = _lookup(idx.reshape(nw, n_ch, ch), table, ch, nbuf)
    out = out.reshape(b0, b1, d)
    return jlayout.with_layout_constraint(
        out, jlayout.Layout(major_to_minor=(0, 1, 2)))
